# baseline (speedup 1.0000x reference)
import dataclasses
import functools

import jax
import jax.numpy as jnp
from jax import lax
from jax.experimental import pallas as pl
from jax.experimental.pallas import tpu as pltpu
from jax.experimental.pallas import tpu_sc as plsc

_N = 10000
_E = 320000
_F = 128
_H = 128
_C = 40
_CPAD = 64

_NC = 2
_NS = 16
_NW = _NC * _NS
_EPT = _E // _NW
_K = 80
_NCH = 125
_EPAD = _NCH * _K
_NPAD = _N
_RPT = _N // _NS
_ZPT = _NPAD // _NS

_BN = 1000


def _vector_mesh():
    return plsc.VectorSubcoreMesh(core_axis_name="c", subcore_axis_name="s")


def _sc_compiler_params():
    cp = pltpu.CompilerParams()
    fields = pltpu.CompilerParams.__dataclass_fields__
    if "needs_layout_passes" in fields:
        cp = dataclasses.replace(cp, needs_layout_passes=False)
    if "use_tc_tiling_on_sc" in fields:
        cp = dataclasses.replace(cp, use_tc_tiling_on_sc=False)
    return cp


@functools.cache
def _make_deg_kernel():
    @functools.partial(
        pl.kernel,
        out_type=jax.ShapeDtypeStruct((_NW, _N), jnp.float32),
        mesh=_vector_mesh(),
        compiler_params=_sc_compiler_params(),
        scratch_types=[
            pltpu.VMEM((_NCH, _K), jnp.int32),
            pltpu.VMEM((_NPAD,), jnp.float32),
        ],
    )
    def _deg_kernel(dst_hbm, deg_hbm, didx, degloc):
        cid = lax.axis_index("c")
        sid = lax.axis_index("s")
        wid = cid * _NS + sid
        pltpu.sync_copy(dst_hbm.at[wid], didx)

        zeros16 = jnp.zeros((16,), jnp.float32)

        @pl.loop(0, _NPAD // 16)
        def _(i):
            degloc[pl.ds(i * 16, 16)] = zeros16

        ones16 = jnp.ones((16,), jnp.float32)

        @pl.loop(0, _NCH)
        def _(c):
            for j in range(_K // 16):
                idx = didx[c, pl.ds(j * 16, 16)]
                plsc.addupdate_scatter(degloc, [idx], ones16)

        pltpu.sync_copy(degloc.at[pl.ds(0, _N)], deg_hbm.at[wid])

    return _deg_kernel


@functools.cache
def _make_msg_kernel(width, stage_table, dtype=jnp.float32):
    scratch = [
        pltpu.VMEM((_NCH, _K), jnp.int32),
        pltpu.VMEM((_NCH, _K), jnp.int32),
        pltpu.VMEM((_K, width), dtype),
        pltpu.VMEM((_K, width), dtype),
        pltpu.VMEM_SHARED((_NPAD, width), dtype),
        pltpu.SemaphoreType.DMA,
        pltpu.SemaphoreType.DMA,
    ]
    if stage_table:
        scratch.append(pltpu.VMEM_SHARED((_N, width), dtype))

    @functools.partial(
        pl.kernel,
        out_type=jax.ShapeDtypeStruct((_NC, _N, width), dtype),
        mesh=_vector_mesh(),
        compiler_params=_sc_compiler_params(),
        scratch_types=scratch,
    )
    def _msg(g_hbm, src_hbm, dst_hbm, zeros_hbm, out_hbm,
             sidx, didx, rows_a, rows_b, acc, sem_a, sem_b, *maybe_tab):
        cid = lax.axis_index("c")
        sid = lax.axis_index("s")
        wid = cid * _NS + sid
        pltpu.sync_copy(src_hbm.at[wid], sidx)
        pltpu.sync_copy(dst_hbm.at[wid], didx)
        pltpu.sync_copy(zeros_hbm, acc.at[pl.ds(sid * _ZPT, _ZPT)])
        if stage_table:
            gtab = maybe_tab[0]
            pltpu.sync_copy(g_hbm.at[pl.ds(sid * _RPT, _RPT)],
                            gtab.at[pl.ds(sid * _RPT, _RPT)])
            g_src = gtab
        else:
            g_src = g_hbm
        plsc.subcore_barrier()

        def gather_start(c, buf, sem):
            pltpu.async_copy(g_src.at[sidx.at[c]], buf, sem)

        def gather_wait(c, buf, sem):
            pltpu.make_async_copy(g_src.at[sidx.at[c]], buf, sem).wait()

        def scatter(c, buf):
            pltpu.sync_copy(buf, acc.at[didx.at[c]], add=True)

        gather_start(0, rows_a, sem_a)

        @pl.loop(0, (_NCH - 1) // 2)
        def _(i):
            c0 = 2 * i
            gather_wait(c0, rows_a, sem_a)
            gather_start(c0 + 1, rows_b, sem_b)
            scatter(c0, rows_a)
            gather_wait(c0 + 1, rows_b, sem_b)
            gather_start(c0 + 2, rows_a, sem_a)
            scatter(c0 + 1, rows_b)

        gather_wait(_NCH - 1, rows_a, sem_a)
        scatter(_NCH - 1, rows_a)

        plsc.subcore_barrier()
        pltpu.sync_copy(
            acc.at[pl.ds(sid * _RPT, _RPT)],
            out_hbm.at[cid, pl.ds(sid * _RPT, _RPT)],
        )

    return _msg


def _dinv_of(degp_ref):
    deg = jnp.sum(degp_ref[0], axis=0) + 1.0
    return lax.rsqrt(deg)


def _prep1_body(x_ref, w1_ref, degp_ref, g1_ref):
    dinv = _dinv_of(degp_ref)
    h = jnp.dot(x_ref[...], w1_ref[...], preferred_element_type=jnp.float32)
    g1_ref[...] = (h * dinv[:, None]).astype(g1_ref.dtype)


def _prep1(x, w1, degp):
    return pl.pallas_call(
        _prep1_body,
        grid=(_N // _BN,),
        in_specs=[
            pl.BlockSpec((_BN, _F), lambda i: (i, 0)),
            pl.BlockSpec((_F, _H), lambda i: (0, 0)),
            pl.BlockSpec((1, _NW, _BN), lambda i: (i, 0, 0)),
        ],
        out_specs=pl.BlockSpec((_BN, _H), lambda i: (i, 0)),
        out_shape=jax.ShapeDtypeStruct((_N, _H), jnp.float32),
    )(x, w1, degp)


def _mid_body(acc_ref, g1_ref, degp_ref, b1_ref, w2p_ref, g2_ref):
    dinv = _dinv_of(degp_ref)
    s32 = (acc_ref[0].astype(jnp.float32) + acc_ref[1].astype(jnp.float32)
           + g1_ref[...].astype(jnp.float32))
    z = s32 * dinv[:, None] + b1_ref[...]
    h2 = jnp.maximum(z, 0.0)
    g2 = jnp.dot(h2, w2p_ref[...], preferred_element_type=jnp.float32)
    g2_ref[...] = (g2 * dinv[:, None]).astype(g2_ref.dtype)


def _mid(acc1, g1, degp, b1r, w2p):
    return pl.pallas_call(
        _mid_body,
        grid=(_N // _BN,),
        in_specs=[
            pl.BlockSpec((_NC, _BN, _H), lambda i: (0, i, 0)),
            pl.BlockSpec((_BN, _H), lambda i: (i, 0)),
            pl.BlockSpec((1, _NW, _BN), lambda i: (i, 0, 0)),
            pl.BlockSpec((1, _H), lambda i: (0, 0)),
            pl.BlockSpec((_H, _CPAD), lambda i: (0, 0)),
        ],
        out_specs=pl.BlockSpec((_BN, _CPAD), lambda i: (i, 0)),
        out_shape=jax.ShapeDtypeStruct((_N, _CPAD), jnp.float32),
    )(acc1, g1, degp, b1r, w2p)


def _fin_body(acc_ref, g2_ref, degp_ref, b2_ref, out_ref):
    dinv = _dinv_of(degp_ref)
    s32 = (acc_ref[0].astype(jnp.float32) + acc_ref[1].astype(jnp.float32)
           + g2_ref[...].astype(jnp.float32))
    z = s32 * dinv[:, None] + b2_ref[...]
    col = lax.broadcasted_iota(jnp.int32, (_BN, _CPAD), 1)
    zm = jnp.where(col < _C, z, -1e30)
    m = jnp.max(zm, axis=1, keepdims=True)
    lse = jnp.log(jnp.sum(jnp.exp(zm - m), axis=1, keepdims=True)) + m
    out_ref[...] = (zm - lse)[:, :_C]


def _fin(acc2, g2, degp, b2p):
    return pl.pallas_call(
        _fin_body,
        grid=(_N // _BN,),
        in_specs=[
            pl.BlockSpec((_NC, _BN, _CPAD), lambda i: (0, i, 0)),
            pl.BlockSpec((_BN, _CPAD), lambda i: (i, 0)),
            pl.BlockSpec((1, _NW, _BN), lambda i: (i, 0, 0)),
            pl.BlockSpec((1, _CPAD), lambda i: (0, 0)),
        ],
        out_specs=pl.BlockSpec((_BN, _C), lambda i: (i, 0)),
        out_shape=jax.ShapeDtypeStruct((_N, _C), jnp.float32),
    )(acc2, g2, degp, b2p)


def kernel(x, edge_index, W1, b1, W2, b2):
    pad = _EPAD - _EPT
    src3 = jnp.pad(edge_index[0].reshape(_NW, _EPT),
                   ((0, 0), (0, pad))).reshape(_NW, _NCH, _K)
    dst3 = jnp.pad(edge_index[1].reshape(_NW, _EPT),
                   ((0, 0), (0, pad)),
                   constant_values=_N).reshape(_NW, _NCH, _K)
    zeros_h = jnp.zeros((_ZPT, _H), jnp.float32)
    zeros_c = jnp.zeros((_ZPT, _CPAD), jnp.float32)
    b1r = b1.reshape(1, _H)
    w2p = jnp.zeros((_H, _CPAD), jnp.float32).at[:, :_C].set(W2)
    b2p = jnp.zeros((1, _CPAD), jnp.float32).at[0, :_C].set(b2)

    degp = _make_deg_kernel()(dst3)
    degp = degp.reshape(_NW, _N // _BN, _BN).transpose(1, 0, 2)
    g1 = _prep1(x, W1, degp)
    acc1 = _make_msg_kernel(_H, False)(g1, src3, dst3, zeros_h)
    g2 = _mid(acc1, g1, degp, b1r, w2p)
    acc2 = _make_msg_kernel(_CPAD, True)(g2, src3, dst3, zeros_c)
    return _fin(acc2, g2, degp, b2p)


# speedup vs baseline: 1.0012x; 1.0012x over previous
import dataclasses
import functools

import jax
import jax.numpy as jnp
from jax import lax
from jax.experimental import pallas as pl
from jax.experimental.pallas import tpu as pltpu
from jax.experimental.pallas import tpu_sc as plsc

_N = 10000
_E = 320000
_F = 128
_H = 128
_C = 40
_CPAD = 64

_NC = 2
_NS = 16
_NW = _NC * _NS
_EPT = _E // _NW
_K = 80
_NCH = 125
_EPAD = _NCH * _K
_NPAD = _N
_RPT = _N // _NS
_ZPT = _NPAD // _NS

_BN = 1000


def _vector_mesh():
    return plsc.VectorSubcoreMesh(core_axis_name="c", subcore_axis_name="s")


def _sc_compiler_params():
    cp = pltpu.CompilerParams()
    fields = pltpu.CompilerParams.__dataclass_fields__
    if "needs_layout_passes" in fields:
        cp = dataclasses.replace(cp, needs_layout_passes=False)
    if "use_tc_tiling_on_sc" in fields:
        cp = dataclasses.replace(cp, use_tc_tiling_on_sc=False)
    return cp


@functools.cache
def _make_deg_kernel():
    @functools.partial(
        pl.kernel,
        out_type=jax.ShapeDtypeStruct((_N // _BN, _NW, _BN), jnp.float32),
        mesh=_vector_mesh(),
        compiler_params=_sc_compiler_params(),
        scratch_types=[
            pltpu.VMEM((_NCH, _K), jnp.int32),
            pltpu.VMEM((_NPAD,), jnp.float32),
        ],
    )
    def _deg_kernel(dst_hbm, deg_hbm, didx, degloc):
        cid = lax.axis_index("c")
        sid = lax.axis_index("s")
        wid = cid * _NS + sid
        pltpu.sync_copy(dst_hbm.at[wid], didx)

        zeros16 = jnp.zeros((16,), jnp.float32)

        @pl.loop(0, _NPAD // 16)
        def _(i):
            degloc[pl.ds(i * 16, 16)] = zeros16

        ones16 = jnp.ones((16,), jnp.float32)

        @pl.loop(0, _NCH)
        def _(c):
            for j in range(_K // 16):
                idx = didx[c, pl.ds(j * 16, 16)]
                plsc.addupdate_scatter(degloc, [idx], ones16)

        for b in range(_N // _BN):
            pltpu.sync_copy(degloc.at[pl.ds(b * _BN, _BN)], deg_hbm.at[b, wid])

    return _deg_kernel


@functools.cache
def _make_msg_kernel(width, stage_table, dtype=jnp.float32):
    scratch = [
        pltpu.VMEM((_NCH, _K), jnp.int32),
        pltpu.VMEM((_NCH, _K), jnp.int32),
        pltpu.VMEM((_K, width), dtype),
        pltpu.VMEM((_K, width), dtype),
        pltpu.VMEM_SHARED((_NPAD, width), dtype),
        pltpu.SemaphoreType.DMA,
        pltpu.SemaphoreType.DMA,
    ]
    if stage_table:
        scratch.append(pltpu.VMEM_SHARED((_N, width), dtype))

    @functools.partial(
        pl.kernel,
        out_type=jax.ShapeDtypeStruct((_NC, _N, width), dtype),
        mesh=_vector_mesh(),
        compiler_params=_sc_compiler_params(),
        scratch_types=scratch,
    )
    def _msg(g_hbm, src_hbm, dst_hbm, zeros_hbm, out_hbm,
             sidx, didx, rows_a, rows_b, acc, sem_a, sem_b, *maybe_tab):
        cid = lax.axis_index("c")
        sid = lax.axis_index("s")
        wid = cid * _NS + sid
        pltpu.sync_copy(src_hbm.at[wid], sidx)
        pltpu.sync_copy(dst_hbm.at[wid], didx)
        pltpu.sync_copy(zeros_hbm, acc.at[pl.ds(sid * _ZPT, _ZPT)])
        if stage_table:
            gtab = maybe_tab[0]
            pltpu.sync_copy(g_hbm.at[pl.ds(sid * _RPT, _RPT)],
                            gtab.at[pl.ds(sid * _RPT, _RPT)])
            g_src = gtab
        else:
            g_src = g_hbm
        plsc.subcore_barrier()

        def gather_start(c, buf, sem):
            pltpu.async_copy(g_src.at[sidx.at[c]], buf, sem)

        def gather_wait(c, buf, sem):
            pltpu.make_async_copy(g_src.at[sidx.at[c]], buf, sem).wait()

        def scatter(c, buf):
            pltpu.sync_copy(buf, acc.at[didx.at[c]], add=True)

        gather_start(0, rows_a, sem_a)

        @pl.loop(0, (_NCH - 1) // 2)
        def _(i):
            c0 = 2 * i
            gather_wait(c0, rows_a, sem_a)
            gather_start(c0 + 1, rows_b, sem_b)
            scatter(c0, rows_a)
            gather_wait(c0 + 1, rows_b, sem_b)
            gather_start(c0 + 2, rows_a, sem_a)
            scatter(c0 + 1, rows_b)

        gather_wait(_NCH - 1, rows_a, sem_a)
        scatter(_NCH - 1, rows_a)

        plsc.subcore_barrier()
        pltpu.sync_copy(
            acc.at[pl.ds(sid * _RPT, _RPT)],
            out_hbm.at[cid, pl.ds(sid * _RPT, _RPT)],
        )

    return _msg


def _dinv_of(degp_ref):
    deg = jnp.sum(degp_ref[0], axis=0) + 1.0
    return lax.rsqrt(deg)


def _prep1_body(x_ref, w1_ref, degp_ref, g1_ref):
    dinv = _dinv_of(degp_ref)
    h = jnp.dot(x_ref[...], w1_ref[...], preferred_element_type=jnp.float32)
    g1_ref[...] = (h * dinv[:, None]).astype(g1_ref.dtype)


def _prep1(x, w1, degp):
    return pl.pallas_call(
        _prep1_body,
        grid=(_N // _BN,),
        in_specs=[
            pl.BlockSpec((_BN, _F), lambda i: (i, 0)),
            pl.BlockSpec((_F, _H), lambda i: (0, 0)),
            pl.BlockSpec((1, _NW, _BN), lambda i: (i, 0, 0)),
        ],
        out_specs=pl.BlockSpec((_BN, _H), lambda i: (i, 0)),
        out_shape=jax.ShapeDtypeStruct((_N, _H), jnp.float32),
    )(x, w1, degp)


def _mid_body(acc_ref, g1_ref, degp_ref, b1_ref, w2p_ref, g2_ref):
    dinv = _dinv_of(degp_ref)
    s32 = (acc_ref[0].astype(jnp.float32) + acc_ref[1].astype(jnp.float32)
           + g1_ref[...].astype(jnp.float32))
    z = s32 * dinv[:, None] + b1_ref[...]
    h2 = jnp.maximum(z, 0.0)
    g2 = jnp.dot(h2, w2p_ref[...], preferred_element_type=jnp.float32)
    g2_ref[...] = (g2 * dinv[:, None]).astype(g2_ref.dtype)


def _mid(acc1, g1, degp, b1r, w2p):
    return pl.pallas_call(
        _mid_body,
        grid=(_N // _BN,),
        in_specs=[
            pl.BlockSpec((_NC, _BN, _H), lambda i: (0, i, 0)),
            pl.BlockSpec((_BN, _H), lambda i: (i, 0)),
            pl.BlockSpec((1, _NW, _BN), lambda i: (i, 0, 0)),
            pl.BlockSpec((1, _H), lambda i: (0, 0)),
            pl.BlockSpec((_H, _CPAD), lambda i: (0, 0)),
        ],
        out_specs=pl.BlockSpec((_BN, _CPAD), lambda i: (i, 0)),
        out_shape=jax.ShapeDtypeStruct((_N, _CPAD), jnp.float32),
    )(acc1, g1, degp, b1r, w2p)


def _fin_body(acc_ref, g2_ref, degp_ref, b2_ref, out_ref):
    dinv = _dinv_of(degp_ref)
    s32 = (acc_ref[0].astype(jnp.float32) + acc_ref[1].astype(jnp.float32)
           + g2_ref[...].astype(jnp.float32))
    z = s32 * dinv[:, None] + b2_ref[...]
    col = lax.broadcasted_iota(jnp.int32, (_BN, _CPAD), 1)
    zm = jnp.where(col < _C, z, -1e30)
    m = jnp.max(zm, axis=1, keepdims=True)
    lse = jnp.log(jnp.sum(jnp.exp(zm - m), axis=1, keepdims=True)) + m
    out_ref[...] = (zm - lse)[:, :_C]


def _fin(acc2, g2, degp, b2p):
    return pl.pallas_call(
        _fin_body,
        grid=(_N // _BN,),
        in_specs=[
            pl.BlockSpec((_NC, _BN, _CPAD), lambda i: (0, i, 0)),
            pl.BlockSpec((_BN, _CPAD), lambda i: (i, 0)),
            pl.BlockSpec((1, _NW, _BN), lambda i: (i, 0, 0)),
            pl.BlockSpec((1, _CPAD), lambda i: (0, 0)),
        ],
        out_specs=pl.BlockSpec((_BN, _C), lambda i: (i, 0)),
        out_shape=jax.ShapeDtypeStruct((_N, _C), jnp.float32),
    )(acc2, g2, degp, b2p)


def kernel(x, edge_index, W1, b1, W2, b2):
    pad = _EPAD - _EPT
    src3 = jnp.pad(edge_index[0].reshape(_NW, _EPT),
                   ((0, 0), (0, pad))).reshape(_NW, _NCH, _K)
    dst3 = jnp.pad(edge_index[1].reshape(_NW, _EPT),
                   ((0, 0), (0, pad)),
                   constant_values=_N).reshape(_NW, _NCH, _K)
    zeros_h = jnp.zeros((_ZPT, _H), jnp.float32)
    zeros_c = jnp.zeros((_ZPT, _CPAD), jnp.float32)
    b1r = b1.reshape(1, _H)
    w2p = jnp.zeros((_H, _CPAD), jnp.float32).at[:, :_C].set(W2)
    b2p = jnp.zeros((1, _CPAD), jnp.float32).at[0, :_C].set(b2)

    degp = _make_deg_kernel()(dst3)
    g1 = _prep1(x, W1, degp)
    acc1 = _make_msg_kernel(_H, False)(g1, src3, dst3, zeros_h)
    g2 = _mid(acc1, g1, degp, b1r, w2p)
    acc2 = _make_msg_kernel(_CPAD, True)(g2, src3, dst3, zeros_c)
    return _fin(acc2, g2, degp, b2p)
